# baseline (speedup 1.0000x reference)
import jax
import jax.numpy as jnp
from jax import lax
from jax.experimental import pallas as pl
from jax.experimental.pallas import tpu as pltpu
from jax.experimental.pallas import tpu_sc as plsc

_NUM_STEPS = 1000
_B, _L, _D = 64, 4096, 128
_TL = 4096
_NB = 2
_LANES = 128
_B_SPLIT = 48


def _make_tables_np():
    import numpy as np

    betas = np.linspace(np.float32(0.0001), np.float32(0.02), _NUM_STEPS,
                        dtype=np.float32)
    alphas = (np.float32(1.0) - betas).astype(np.float32)
    alphas_bar = np.cumprod(alphas, dtype=np.float32)
    sa = np.sqrt(alphas_bar, dtype=np.float32)
    sb = np.sqrt((np.float32(1.0) - alphas_bar).astype(np.float32),
                 dtype=np.float32)
    packed = np.zeros((_NUM_STEPS, _LANES), dtype=np.float32)
    packed[:, 0] = sa
    packed[:, 1] = sb
    return sa, sb, packed


def _sc_gather_body(table_hbm, t_hbm, out_hbm, idx_v, rows_v, sem):
    wid = lax.axis_index("s") * 2 + lax.axis_index("c")

    @pl.when(wid == 0)
    def _():
        pltpu.sync_copy(t_hbm, idx_v)
        pltpu.async_copy(table_hbm.at[idx_v], rows_v, sem).wait()
        pltpu.sync_copy(rows_v, out_hbm)


def _sc_gather(table, t):
    mesh = plsc.VectorSubcoreMesh(core_axis_name="c", subcore_axis_name="s")
    return pl.kernel(
        _sc_gather_body,
        mesh=mesh,
        out_type=jax.ShapeDtypeStruct((_B, _LANES), jnp.float32),
        scratch_types=[
            pltpu.VMEM((_B,), jnp.int32),
            pltpu.VMEM((_B, _LANES), jnp.float32),
            pltpu.SemaphoreType.DMA,
        ],
    )(table, t)


def _blend_lo_body(t_ref, sa_tab_ref, sb_tab_ref, x_ref, n_ref, o_ref):
    b = pl.program_id(0)
    for i in range(_NB):
        tb = t_ref[b * _NB + i]
        sa = sa_tab_ref[tb]
        sb = sb_tab_ref[tb]
        o_ref[i] = sa * x_ref[i] + sb * n_ref[i]


def _blend_hi_body(sab_ref, x_ref, n_ref, prev_ref, o_ref):
    b = pl.program_id(0)
    for i in range(_NB):
        r = _B_SPLIT + b * _NB + i
        sa = sab_ref[r, 0]
        sb = sab_ref[r, 1]
        o_ref[i] = sa * x_ref[i] + sb * n_ref[i]


def kernel(x, noise, t):
    t = t.astype(jnp.int32)
    sa_np, sb_np, packed_np = _make_tables_np()
    sa_tab = jnp.asarray(sa_np)
    sb_tab = jnp.asarray(sb_np)
    table = jnp.asarray(packed_np)
    sab = _sc_gather(table, t)

    smem = pl.BlockSpec(memory_space=pltpu.SMEM)
    big = pl.BlockSpec((_NB, _TL, _D), lambda b: (b, 0, 0))
    out_lo = pl.pallas_call(
        _blend_lo_body,
        grid=(_B_SPLIT // _NB,),
        in_specs=[smem, smem, smem, big, big],
        out_specs=big,
        out_shape=jax.ShapeDtypeStruct((_B, _L, _D), jnp.float32),
    )(t, sa_tab, sb_tab, x, noise)

    hi_blocks = _B_SPLIT // _NB
    big_hi = pl.BlockSpec((_NB, _TL, _D), lambda b: (hi_blocks + b, 0, 0))
    anyspec = pl.BlockSpec(memory_space=pltpu.MemorySpace.HBM)
    return pl.pallas_call(
        _blend_hi_body,
        grid=((_B - _B_SPLIT) // _NB,),
        in_specs=[smem, big_hi, big_hi, anyspec],
        out_specs=big_hi,
        out_shape=jax.ShapeDtypeStruct((_B, _L, _D), jnp.float32),
        input_output_aliases={3: 0},
    )(sab, x, noise, out_lo)


# speedup vs baseline: 1.1331x; 1.1331x over previous
import jax
import jax.numpy as jnp
from jax import lax
from jax.experimental import pallas as pl
from jax.experimental.pallas import tpu as pltpu
from jax.experimental.pallas import tpu_sc as plsc

_NUM_STEPS = 1000
_B, _L, _D = 64, 4096, 128
_TL = 4096
_NB = 2
_LANES = 128
_B_SPLIT = 48


def _make_tables_np():
    import numpy as np

    betas = np.linspace(np.float32(0.0001), np.float32(0.02), _NUM_STEPS,
                        dtype=np.float32)
    alphas = (np.float32(1.0) - betas).astype(np.float32)
    alphas_bar = np.cumprod(alphas, dtype=np.float32)
    sa = np.sqrt(alphas_bar, dtype=np.float32)
    sb = np.sqrt((np.float32(1.0) - alphas_bar).astype(np.float32),
                 dtype=np.float32)
    packed = np.zeros((_NUM_STEPS, _LANES), dtype=np.float32)
    packed[:, 0] = sa
    packed[:, 1] = sb
    return sa, sb, packed


def _sc_gather_body(table_hbm, t_hbm, out_hbm, idx_v, rows_v, sem):
    wid = lax.axis_index("s") * 2 + lax.axis_index("c")

    @pl.when(wid == 0)
    def _():
        pltpu.sync_copy(t_hbm, idx_v)
        pltpu.async_copy(table_hbm.at[idx_v], rows_v, sem).wait()
        pltpu.sync_copy(rows_v, out_hbm)


def _sc_gather(table, t):
    mesh = plsc.VectorSubcoreMesh(core_axis_name="c", subcore_axis_name="s")
    return pl.kernel(
        _sc_gather_body,
        mesh=mesh,
        out_type=jax.ShapeDtypeStruct((_B, _LANES), jnp.float32),
        scratch_types=[
            pltpu.VMEM((_B,), jnp.int32),
            pltpu.VMEM((_B, _LANES), jnp.float32),
            pltpu.SemaphoreType.DMA,
        ],
    )(table, t)


def _blend_lo_body(t_ref, sa_tab_ref, sb_tab_ref, x_ref, n_ref, o_ref):
    b = pl.program_id(0)
    for i in range(_NB):
        tb = t_ref[b * _NB + i]
        sa = sa_tab_ref[tb]
        sb = sb_tab_ref[tb]
        o_ref[i] = sa * x_ref[i] + sb * n_ref[i]


def _blend_hi_body(t_ref, sa_tab_ref, sb_tab_ref, x_ref, n_ref, prev_ref, o_ref):
    b = pl.program_id(0)
    for i in range(_NB):
        r = _B_SPLIT + b * _NB + i
        tb = t_ref[r]
        sa = sa_tab_ref[tb]
        sb = sb_tab_ref[tb]
        o_ref[i] = sa * x_ref[i] + sb * n_ref[i]


def kernel(x, noise, t):
    t = t.astype(jnp.int32)
    sa_np, sb_np, packed_np = _make_tables_np()
    sa_tab = jnp.asarray(sa_np)
    sb_tab = jnp.asarray(sb_np)
    table = jnp.asarray(packed_np)
    del table

    smem = pl.BlockSpec(memory_space=pltpu.SMEM)
    big = pl.BlockSpec((_NB, _TL, _D), lambda b: (b, 0, 0))
    out_lo = pl.pallas_call(
        _blend_lo_body,
        grid=(_B_SPLIT // _NB,),
        in_specs=[smem, smem, smem, big, big],
        out_specs=big,
        out_shape=jax.ShapeDtypeStruct((_B, _L, _D), jnp.float32),
    )(t, sa_tab, sb_tab, x, noise)

    hi_blocks = _B_SPLIT // _NB
    big_hi = pl.BlockSpec((_NB, _TL, _D), lambda b: (hi_blocks + b, 0, 0))
    anyspec = pl.BlockSpec(memory_space=pltpu.MemorySpace.HBM)
    return pl.pallas_call(
        _blend_hi_body,
        grid=((_B - _B_SPLIT) // _NB,),
        in_specs=[smem, smem, smem, big_hi, big_hi, anyspec],
        out_specs=big_hi,
        out_shape=jax.ShapeDtypeStruct((_B, _L, _D), jnp.float32),
        input_output_aliases={5: 0},
    )(t, sa_tab, sb_tab, x, noise, out_lo)


# speedup vs baseline: 1.1511x; 1.0158x over previous
import jax
import jax.numpy as jnp
from jax import lax
from jax.experimental import pallas as pl
from jax.experimental.pallas import tpu as pltpu
from jax.experimental.pallas import tpu_sc as plsc

_NUM_STEPS = 1000
_B, _L, _D = 64, 4096, 128
_TL = 4096
_NB = 2
_LANES = 128
_B_SPLIT = 64


def _make_tables_np():
    import numpy as np

    betas = np.linspace(np.float32(0.0001), np.float32(0.02), _NUM_STEPS,
                        dtype=np.float32)
    alphas = (np.float32(1.0) - betas).astype(np.float32)
    alphas_bar = np.cumprod(alphas, dtype=np.float32)
    sa = np.sqrt(alphas_bar, dtype=np.float32)
    sb = np.sqrt((np.float32(1.0) - alphas_bar).astype(np.float32),
                 dtype=np.float32)
    packed = np.zeros((_NUM_STEPS, _LANES), dtype=np.float32)
    packed[:, 0] = sa
    packed[:, 1] = sb
    return sa, sb, packed


def _sc_gather_body(table_hbm, t_hbm, out_hbm, idx_v, rows_v, sem):
    wid = lax.axis_index("s") * 2 + lax.axis_index("c")

    @pl.when(wid == 0)
    def _():
        pltpu.sync_copy(t_hbm, idx_v)
        pltpu.async_copy(table_hbm.at[idx_v], rows_v, sem).wait()
        pltpu.sync_copy(rows_v, out_hbm)


def _sc_gather(table, t):
    mesh = plsc.VectorSubcoreMesh(core_axis_name="c", subcore_axis_name="s")
    return pl.kernel(
        _sc_gather_body,
        mesh=mesh,
        out_type=jax.ShapeDtypeStruct((_B, _LANES), jnp.float32),
        scratch_types=[
            pltpu.VMEM((_B,), jnp.int32),
            pltpu.VMEM((_B, _LANES), jnp.float32),
            pltpu.SemaphoreType.DMA,
        ],
    )(table, t)


def _blend_lo_body(t_ref, sa_tab_ref, sb_tab_ref, x_ref, n_ref, o_ref):
    b = pl.program_id(0)
    for i in range(_NB):
        tb = t_ref[b * _NB + i]
        sa = sa_tab_ref[tb]
        sb = sb_tab_ref[tb]
        o_ref[i] = sa * x_ref[i] + sb * n_ref[i]


def _blend_hi_body(t_ref, sa_tab_ref, sb_tab_ref, x_ref, n_ref, prev_ref, o_ref):
    b = pl.program_id(0)
    for i in range(_NB):
        r = _B_SPLIT + b * _NB + i
        tb = t_ref[r]
        sa = sa_tab_ref[tb]
        sb = sb_tab_ref[tb]
        o_ref[i] = sa * x_ref[i] + sb * n_ref[i]


def kernel(x, noise, t):
    t = t.astype(jnp.int32)
    sa_np, sb_np, packed_np = _make_tables_np()
    sa_tab = jnp.asarray(sa_np)
    sb_tab = jnp.asarray(sb_np)
    table = jnp.asarray(packed_np)
    del table

    smem = pl.BlockSpec(memory_space=pltpu.SMEM)
    big = pl.BlockSpec((_NB, _TL, _D), lambda b: (b, 0, 0))
    out_lo = pl.pallas_call(
        _blend_lo_body,
        grid=(_B_SPLIT // _NB,),
        in_specs=[smem, smem, smem, big, big],
        out_specs=big,
        out_shape=jax.ShapeDtypeStruct((_B, _L, _D), jnp.float32),
    )(t, sa_tab, sb_tab, x, noise)

    if _B_SPLIT == _B:
        return out_lo
    hi_blocks = _B_SPLIT // _NB
    big_hi = pl.BlockSpec((_NB, _TL, _D), lambda b: (hi_blocks + b, 0, 0))
    anyspec = pl.BlockSpec(memory_space=pltpu.MemorySpace.HBM)
    return pl.pallas_call(
        _blend_hi_body,
        grid=((_B - _B_SPLIT) // _NB,),
        in_specs=[smem, smem, smem, big_hi, big_hi, anyspec],
        out_specs=big_hi,
        out_shape=jax.ShapeDtypeStruct((_B, _L, _D), jnp.float32),
        input_output_aliases={5: 0},
    )(t, sa_tab, sb_tab, x, noise, out_lo)
